# Initial kernel scaffold; baseline (speedup 1.0000x reference)
import jax
import jax.numpy as jnp
from jax.experimental import pallas as pl


def kernel(x_user, x_item, edge_index_u2i, edge_index_i2u, edge_index_uu):
    raise NotImplementedError("write your pallas kernel here")


# speedup vs baseline: 5.6499x; 5.6499x over previous
import jax
import jax.numpy as jnp
from jax import lax
from jax.experimental import pallas as pl
from jax.experimental.pallas import tpu as pltpu
from jax.experimental.pallas import tpu_sc as plsc

N_USER = 10000
N_ITEM = 10000
D = 128
E = 160000
NS = 16
CHUNK = 256
IDX_ROWS = CHUNK // 128
CHUNKS_PER_TILE = 40
EP = NS * CHUNKS_PER_TILE * CHUNK
ROWS_PER_TILE = EP // 128 // NS
PAD = 10112
TILE_ROWS = PAD // NS


def _body(xu, xi, su2i, du2i, si2u, di2u, suu, duu,
          out_user, out_item,
          acc, idx_s, idx_d, rows, sem):
    c = lax.axis_index("c")
    s = lax.axis_index("s")
    base = s * TILE_ROWS

    zeros16 = jnp.zeros((16,), jnp.float32)

    def zero_row(i, carry):
        for q in range(D // 16):
            rows[i, pl.ds(q * 16, 16)] = zeros16
        return carry

    lax.fori_loop(0, CHUNK, zero_row, 0)
    pltpu.sync_copy(rows.at[pl.ds(0, CHUNK)], acc.at[pl.ds(base, CHUNK)])
    pltpu.sync_copy(rows.at[pl.ds(0, TILE_ROWS - CHUNK)],
                    acc.at[pl.ds(base + CHUNK, TILE_ROWS - CHUNK)])
    plsc.subcore_barrier()

    def run_type(table, src_rows, dst_rows):
        def chunk_body(i, carry):
            r0 = s * ROWS_PER_TILE + i * IDX_ROWS
            pltpu.sync_copy(src_rows.at[pl.ds(r0, IDX_ROWS)], idx_s)
            pltpu.sync_copy(dst_rows.at[pl.ds(r0, IDX_ROWS)], idx_d)
            handles = [
                pltpu.async_copy(table.at[idx_s.at[j]],
                                 rows.at[pl.ds(j * 128, 128)], sem)
                for j in range(IDX_ROWS)
            ]
            for h in handles:
                h.wait()
            for j in range(IDX_ROWS):
                pltpu.sync_copy(rows.at[pl.ds(j * 128, 128)],
                                acc.at[idx_d.at[j]], add=True)
            return carry

        lax.fori_loop(0, CHUNKS_PER_TILE, chunk_body, 0)

    @pl.when(c == 0)
    def _():
        run_type(xu, su2i, du2i)
        plsc.subcore_barrier()
        pltpu.sync_copy(acc.at[pl.ds(base, TILE_ROWS)],
                        out_item.at[pl.ds(base, TILE_ROWS)])

    @pl.when(c == 1)
    def _():
        run_type(xi, si2u, di2u)
        run_type(xu, suu, duu)
        plsc.subcore_barrier()
        pltpu.sync_copy(acc.at[pl.ds(base, TILE_ROWS)],
                        out_user.at[pl.ds(base, TILE_ROWS)])


def _make_sc_call():
    mesh = plsc.VectorSubcoreMesh(core_axis_name="c", subcore_axis_name="s",
                                  num_cores=2, num_subcores=NS)
    return pl.kernel(
        _body,
        out_type=[jax.ShapeDtypeStruct((PAD, D), jnp.float32)] * 2,
        mesh=mesh,
        scratch_types=[
            pltpu.VMEM_SHARED((PAD, D), jnp.float32),
            pltpu.VMEM((IDX_ROWS, 128), jnp.int32),
            pltpu.VMEM((IDX_ROWS, 128), jnp.int32),
            pltpu.VMEM((CHUNK, D), jnp.float32),
            pltpu.SemaphoreType.DMA,
        ],
    )


_sc_call_cache = None


def _get_sc_call():
    global _sc_call_cache
    if _sc_call_cache is None:
        _sc_call_cache = _make_sc_call()
    return _sc_call_cache


def _pad_split(ei):
    ei = ei.astype(jnp.int32)
    npad = EP - E
    r = jnp.arange(npad, dtype=jnp.int32)
    src = jnp.concatenate([ei[0], r % 16]).reshape(EP // 128, 128)
    dst = jnp.concatenate([ei[1], N_USER + (r % (PAD - N_USER))]).reshape(
        EP // 128, 128)
    return src, dst


@jax.jit
def kernel(x_user, x_item, edge_index_u2i, edge_index_i2u, edge_index_uu):
    su2i, du2i = _pad_split(edge_index_u2i)
    si2u, di2u = _pad_split(edge_index_i2u)
    suu, duu = _pad_split(edge_index_uu)
    ou, oi = _get_sc_call()(
        x_user, x_item, su2i, du2i, si2u, di2u, suu, duu)
    return (ou[:N_USER], oi[:N_ITEM])


# speedup vs baseline: 5.6523x; 1.0004x over previous
import jax
import jax.numpy as jnp
from jax import lax
from jax.experimental import pallas as pl
from jax.experimental.pallas import tpu as pltpu
from jax.experimental.pallas import tpu_sc as plsc

N_USER = 10000
N_ITEM = 10000
D = 128
E = 160000
NS = 16
CHUNK = 256
IDX_ROWS = CHUNK // 128
CHUNKS_PER_TILE = 40
EP = NS * CHUNKS_PER_TILE * CHUNK
ROWS_PER_TILE = EP // 128 // NS
PAD = 10112
TILE_ROWS = PAD // NS


def _body(xu, xi, su2i, du2i, si2u, di2u, suu, duu,
          out_user, out_item,
          acc, idx_s, idx_d, rows, sem):
    c = lax.axis_index("c")
    s = lax.axis_index("s")
    base = s * TILE_ROWS

    zeros16 = jnp.zeros((16,), jnp.float32)

    def zero_row(i, carry):
        for q in range(D // 16):
            rows[i, pl.ds(q * 16, 16)] = zeros16
        return carry

    lax.fori_loop(0, CHUNK, zero_row, 0)
    off = 0
    while off < TILE_ROWS:
        n = min(CHUNK, TILE_ROWS - off)
        pltpu.sync_copy(rows.at[pl.ds(0, n)],
                        acc.at[pl.ds(base + off, n)])
        off += n
    plsc.subcore_barrier()

    def run_type(table, src_rows, dst_rows):
        def chunk_body(i, carry):
            r0 = s * ROWS_PER_TILE + i * IDX_ROWS
            pltpu.sync_copy(src_rows.at[pl.ds(r0, IDX_ROWS)], idx_s)
            pltpu.sync_copy(dst_rows.at[pl.ds(r0, IDX_ROWS)], idx_d)
            handles = [
                pltpu.async_copy(table.at[idx_s.at[j]],
                                 rows.at[pl.ds(j * 128, 128)], sem)
                for j in range(IDX_ROWS)
            ]
            for h in handles:
                h.wait()
            for j in range(IDX_ROWS):
                pltpu.sync_copy(rows.at[pl.ds(j * 128, 128)],
                                acc.at[idx_d.at[j]], add=True)
            return carry

        lax.fori_loop(0, CHUNKS_PER_TILE, chunk_body, 0)

    @pl.when(c == 0)
    def _():
        run_type(xu, su2i, du2i)
        plsc.subcore_barrier()
        pltpu.sync_copy(acc.at[pl.ds(base, TILE_ROWS)],
                        out_item.at[pl.ds(base, TILE_ROWS)])

    @pl.when(c == 1)
    def _():
        run_type(xi, si2u, di2u)
        run_type(xu, suu, duu)
        plsc.subcore_barrier()
        pltpu.sync_copy(acc.at[pl.ds(base, TILE_ROWS)],
                        out_user.at[pl.ds(base, TILE_ROWS)])


def _make_sc_call():
    mesh = plsc.VectorSubcoreMesh(core_axis_name="c", subcore_axis_name="s",
                                  num_cores=2, num_subcores=NS)
    return pl.kernel(
        _body,
        out_type=[jax.ShapeDtypeStruct((PAD, D), jnp.float32)] * 2,
        mesh=mesh,
        scratch_types=[
            pltpu.VMEM_SHARED((PAD, D), jnp.float32),
            pltpu.VMEM((IDX_ROWS, 128), jnp.int32),
            pltpu.VMEM((IDX_ROWS, 128), jnp.int32),
            pltpu.VMEM((CHUNK, D), jnp.float32),
            pltpu.SemaphoreType.DMA,
        ],
    )


_sc_call_cache = None


def _get_sc_call():
    global _sc_call_cache
    if _sc_call_cache is None:
        _sc_call_cache = _make_sc_call()
    return _sc_call_cache


def _pad_split(ei):
    ei = ei.astype(jnp.int32)
    npad = EP - E
    r = jnp.arange(npad, dtype=jnp.int32)
    src = jnp.concatenate([ei[0], r % 16]).reshape(EP // 128, 128)
    dst = jnp.concatenate([ei[1], N_USER + (r % (PAD - N_USER))]).reshape(
        EP // 128, 128)
    return src, dst


@jax.jit
def kernel(x_user, x_item, edge_index_u2i, edge_index_i2u, edge_index_uu):
    su2i, du2i = _pad_split(edge_index_u2i)
    si2u, di2u = _pad_split(edge_index_i2u)
    suu, duu = _pad_split(edge_index_uu)
    ou, oi = _get_sc_call()(
        x_user, x_item, su2i, du2i, si2u, di2u, suu, duu)
    return (ou[:N_USER], oi[:N_ITEM])


# speedup vs baseline: 9.0252x; 1.5967x over previous
import jax
import jax.numpy as jnp
from jax import lax
from jax.experimental import pallas as pl
from jax.experimental.pallas import tpu as pltpu
from jax.experimental.pallas import tpu_sc as plsc

N_USER = 10000
N_ITEM = 10000
D = 128
E = 160000
NS = 16
CHUNK = 128
EP = 163840
ROWS_ALL = EP // 128
ROWS_TILE = ROWS_ALL // 2 // NS
NPAIRS = ROWS_TILE // 2
PAD = 10112
TILE_ROWS = PAD // NS


def _body(xu, xi, su2i, du2i, si2u, di2u, suu, duu,
          pu, pi,
          acc, idx_s, idx_d, buf_a, buf_b, gsem_a, gsem_b):
    c = lax.axis_index("c")
    s = lax.axis_index("s")
    base = s * TILE_ROWS
    row0 = c * (ROWS_ALL // 2) + s * ROWS_TILE

    zeros16 = jnp.zeros((16,), jnp.float32)

    def zero_buf_row(i, carry):
        for q in range(D // 16):
            buf_a[i, pl.ds(q * 16, 16)] = zeros16
        return carry

    def zero_acc():
        lax.fori_loop(0, CHUNK, zero_buf_row, 0)
        off = 0
        while off < TILE_ROWS:
            n = min(CHUNK, TILE_ROWS - off)
            pltpu.sync_copy(buf_a.at[pl.ds(0, n)],
                            acc.at[pl.ds(base + off, n)])
            off += n

    def run_type(table, src_rows, dst_rows):
        pltpu.sync_copy(src_rows.at[pl.ds(row0, ROWS_TILE)], idx_s)
        pltpu.sync_copy(dst_rows.at[pl.ds(row0, ROWS_TILE)], idx_d)
        pltpu.async_copy(table.at[idx_s.at[0]], buf_a, gsem_a)

        def pair(p, carry):
            r_e = 2 * p
            r_o = 2 * p + 1
            pltpu.make_async_copy(table.at[idx_s.at[r_e]],
                                  buf_a, gsem_a).wait()
            pltpu.async_copy(table.at[idx_s.at[r_o]], buf_b, gsem_b)
            pltpu.sync_copy(buf_a, acc.at[idx_d.at[r_e]], add=True)
            pltpu.make_async_copy(table.at[idx_s.at[r_o]],
                                  buf_b, gsem_b).wait()

            @pl.when(p < NPAIRS - 1)
            def _():
                pltpu.async_copy(table.at[idx_s.at[r_e + 2]], buf_a, gsem_a)

            pltpu.sync_copy(buf_b, acc.at[idx_d.at[r_o]], add=True)
            return carry

        lax.fori_loop(0, NPAIRS, pair, 0)

    def writeout(out_ref):
        pltpu.sync_copy(acc.at[pl.ds(base, TILE_ROWS)],
                        out_ref.at[c, pl.ds(base, TILE_ROWS)])

    zero_acc()
    plsc.subcore_barrier()
    run_type(xu, su2i, du2i)
    plsc.subcore_barrier()
    writeout(pi)
    zero_acc()
    plsc.subcore_barrier()
    run_type(xi, si2u, di2u)
    run_type(xu, suu, duu)
    plsc.subcore_barrier()
    writeout(pu)


def _make_sc_call():
    mesh = plsc.VectorSubcoreMesh(core_axis_name="c", subcore_axis_name="s",
                                  num_cores=2, num_subcores=NS)
    return pl.kernel(
        _body,
        out_type=[jax.ShapeDtypeStruct((2, PAD, D), jnp.float32)] * 2,
        mesh=mesh,
        scratch_types=[
            pltpu.VMEM_SHARED((PAD, D), jnp.float32),
            pltpu.VMEM((ROWS_TILE, 128), jnp.int32),
            pltpu.VMEM((ROWS_TILE, 128), jnp.int32),
            pltpu.VMEM((CHUNK, D), jnp.float32),
            pltpu.VMEM((CHUNK, D), jnp.float32),
            pltpu.SemaphoreType.DMA,
            pltpu.SemaphoreType.DMA,
        ],
    )


_sc_call_cache = None


def _get_sc_call():
    global _sc_call_cache
    if _sc_call_cache is None:
        _sc_call_cache = _make_sc_call()
    return _sc_call_cache


def _pad_split(ei):
    ei = ei.astype(jnp.int32)
    npad = EP - E
    r = jnp.arange(npad, dtype=jnp.int32)
    src = jnp.concatenate([ei[0], r % 16]).reshape(ROWS_ALL, 128)
    dst = jnp.concatenate([ei[1], N_USER + (r % (PAD - N_USER))]).reshape(
        ROWS_ALL, 128)
    return src, dst


@jax.jit
def kernel(x_user, x_item, edge_index_u2i, edge_index_i2u, edge_index_uu):
    su2i, du2i = _pad_split(edge_index_u2i)
    si2u, di2u = _pad_split(edge_index_i2u)
    suu, duu = _pad_split(edge_index_uu)
    pu, pi = _get_sc_call()(
        x_user, x_item, su2i, du2i, si2u, di2u, suu, duu)
    out_user = pu[0, :N_USER] + pu[1, :N_USER]
    out_item = pi[0, :N_ITEM] + pi[1, :N_ITEM]
    return (out_user, out_item)
